# baseline (speedup 1.0000x reference)
import functools

import jax
import jax.numpy as jnp
from jax import lax
from jax.experimental import pallas as pl
from jax.experimental.pallas import tpu as pltpu
from jax.experimental.pallas import tpu_sc as plsc

EMB_DIM = 64
BATCH = 4096
SEQ = 200
SA = 128
SB = SEQ - SA

_info = plsc.get_sparse_core_info()
NC, NS, NL = _info.num_cores, _info.num_subcores, _info.num_lanes
NW = NC * NS
BPW = BATCH // NW

G = 4
NGRP = BPW // G
NPAIR = NGRP // 2


def _sc_pool_body(ids_hbm, emb_hbm, sums_hbm,
                  idx0, idx1, ra0, rb0, ra1, rb1, sums_v,
                  semi0, semi1, semg0, semg1):
    wid = lax.axis_index("s") * NC + lax.axis_index("c")
    base = wid * BPW

    def idx_copy(g, idx, semi):
        return pltpu.make_async_copy(
            ids_hbm.at[pl.ds(base + g * G, G)], idx, semi)

    def gather_copies(idx, ra, rb, semg):
        cps = []
        for b in range(G):
            cps.append(pltpu.make_async_copy(
                emb_hbm.at[idx.at[b, pl.ds(0, SA)]], ra.at[b], semg))
            cps.append(pltpu.make_async_copy(
                emb_hbm.at[idx.at[b, pl.ds(SA, SB)]], rb.at[b], semg))
        return cps

    def counts(idx):
        lane = lax.iota(jnp.int32, 16)
        ds = []
        for b in range(G):
            def body_c(r, c):
                nz = idx[b, pl.ds(16 * r, 16)] != 0
                return c + jnp.where(nz, 1.0, 0.0)

            c = lax.fori_loop(0, 12, body_c,
                              jnp.zeros((16,), jnp.float32), unroll=4)
            tail = (idx[b, pl.ds(SEQ - 16, 16)] != 0) & (lane >= 8)
            c = c + jnp.where(tail, 1.0, 0.0)
            ds.append(c)
        return ds

    def accumulate(g, ds, ra, rb):
        for b in range(G):
            def body_a(r, accs):
                return (accs[0] + ra[b, r, pl.ds(0, 16)],
                        accs[1] + ra[b, r, pl.ds(16, 16)],
                        accs[2] + ra[b, r, pl.ds(32, 16)],
                        accs[3] + ra[b, r, pl.ds(48, 16)])

            def body_b(r, accs):
                return (accs[0] + rb[b, r, pl.ds(0, 16)],
                        accs[1] + rb[b, r, pl.ds(16, 16)],
                        accs[2] + rb[b, r, pl.ds(32, 16)],
                        accs[3] + rb[b, r, pl.ds(48, 16)])

            z = jnp.zeros((16,), jnp.float32)
            a = lax.fori_loop(0, SA, body_a, (z, z, z, z), unroll=8)
            a = lax.fori_loop(0, SB, body_b, a, unroll=8)

            slot = g * G + b
            sums_v[slot, pl.ds(0, 16)] = a[0]
            sums_v[slot, pl.ds(16, 16)] = a[1]
            sums_v[slot, pl.ds(32, 16)] = a[2]
            sums_v[slot, pl.ds(48, 16)] = a[3]
            sums_v[slot, pl.ds(64, 16)] = ds[b]

    idx_copy(0, idx0, semi0).start()
    idx_copy(1, idx1, semi1).start()
    idx_copy(0, idx0, semi0).wait()
    for c in gather_copies(idx0, ra0, rb0, semg0):
        c.start()

    def pair_body(i, carry):
        g0 = 2 * i

        idx_copy(g0 + 1, idx1, semi1).wait()
        for c in gather_copies(idx1, ra1, rb1, semg1):
            c.start()
        for c in gather_copies(idx0, ra0, rb0, semg0):
            c.wait()

        ds0 = counts(idx0)

        @pl.when(g0 + 2 < NGRP)
        def _():
            idx_copy(g0 + 2, idx0, semi0).start()

        accumulate(g0, ds0, ra0, rb0)

        @pl.when(g0 + 2 < NGRP)
        def _():
            idx_copy(g0 + 2, idx0, semi0).wait()
            for c in gather_copies(idx0, ra0, rb0, semg0):
                c.start()

        for c in gather_copies(idx1, ra1, rb1, semg1):
            c.wait()

        ds1 = counts(idx1)

        @pl.when(g0 + 3 < NGRP)
        def _():
            idx_copy(g0 + 3, idx1, semi1).start()

        accumulate(g0 + 1, ds1, ra1, rb1)
        return carry

    lax.fori_loop(0, NPAIR, pair_body, 0)
    pltpu.sync_copy(sums_v, sums_hbm.at[pl.ds(base, BPW)])


_sc_pool = functools.partial(
    pl.kernel,
    out_type=jax.ShapeDtypeStruct((BATCH, EMB_DIM + 16), jnp.float32),
    mesh=plsc.VectorSubcoreMesh(core_axis_name="c", subcore_axis_name="s"),
    compiler_params=pltpu.CompilerParams(use_tc_tiling_on_sc=False),
    scratch_types=[
        pltpu.VMEM((G, SEQ), jnp.int32),
        pltpu.VMEM((G, SEQ), jnp.int32),
        pltpu.VMEM((G, SA, EMB_DIM), jnp.float32),
        pltpu.VMEM((G, SB, EMB_DIM), jnp.float32),
        pltpu.VMEM((G, SA, EMB_DIM), jnp.float32),
        pltpu.VMEM((G, SB, EMB_DIM), jnp.float32),
        pltpu.VMEM((BPW, EMB_DIM + 16), jnp.float32),
        pltpu.SemaphoreType.DMA,
        pltpu.SemaphoreType.DMA,
        pltpu.SemaphoreType.DMA,
        pltpu.SemaphoreType.DMA,
    ],
)(_sc_pool_body)


def _mlp_body(sums_ref, w1_ref, b1_ref, w2_ref, b2_ref, out_ref):
    cnt = jnp.sum(sums_ref[:, EMB_DIM:], axis=1, keepdims=True)
    avg = sums_ref[:, :EMB_DIM] / jnp.maximum(cnt, 1e-6)
    h = jnp.dot(avg, w1_ref[...], preferred_element_type=jnp.float32,
                precision=lax.Precision.HIGHEST) + b1_ref[...]
    h = jnp.maximum(h, 0.0)
    out_ref[...] = jnp.dot(h, w2_ref[...], preferred_element_type=jnp.float32,
                           precision=lax.Precision.HIGHEST) + b2_ref[...]


def kernel(ids, emb, W1, b1, W2, b2):
    ids = ids.astype(jnp.int32)
    sums = _sc_pool(ids, emb)

    blk = 512
    grid = (BATCH // blk,)
    hidden = W1.shape[1]
    out_dim = W2.shape[1]
    out = pl.pallas_call(
        _mlp_body,
        grid=grid,
        in_specs=[
            pl.BlockSpec((blk, EMB_DIM + 16), lambda i: (i, 0)),
            pl.BlockSpec((EMB_DIM, hidden), lambda i: (0, 0)),
            pl.BlockSpec((1, hidden), lambda i: (0, 0)),
            pl.BlockSpec((hidden, out_dim), lambda i: (0, 0)),
            pl.BlockSpec((1, out_dim), lambda i: (0, 0)),
        ],
        out_specs=pl.BlockSpec((blk, out_dim), lambda i: (i, 0)),
        out_shape=jax.ShapeDtypeStruct((BATCH, out_dim), jnp.float32),
    )(sums, W1, b1[None, :], W2, b2[None, :])
    return out
